# Initial kernel scaffold; baseline (speedup 1.0000x reference)
import jax
import jax.numpy as jnp
from jax.experimental import pallas as pl


def kernel(x_user, x_item, edge_attr_u2i, edge_attr_i2u, params, edge_index_u2i, edge_index_i2u):
    raise NotImplementedError("write your pallas kernel here")


# speedup vs baseline: 5.8649x; 5.8649x over previous
import jax
import jax.numpy as jnp
from jax.experimental import pallas as pl

N_NODE = 10000
D = 256
H = 8
C = 32
TILE = 256
N_PAD = 10240


def _proj_kernel(x_ref, w_ref, a_ref, h_ref, s_ref):
    h = jnp.dot(x_ref[...], w_ref[...], preferred_element_type=jnp.float32)
    h_ref[...] = h
    s_ref[...] = jnp.dot(h, a_ref[...], preferred_element_type=jnp.float32)


def _proj(x, w, a_mat):
    n, k = x.shape
    h, s = pl.pallas_call(
        _proj_kernel,
        grid=(n // TILE,),
        in_specs=[
            pl.BlockSpec((TILE, k), lambda i: (i, 0)),
            pl.BlockSpec((k, D), lambda i: (0, 0)),
            pl.BlockSpec((D, 128), lambda i: (0, 0)),
        ],
        out_specs=[
            pl.BlockSpec((TILE, D), lambda i: (i, 0)),
            pl.BlockSpec((TILE, 128), lambda i: (i, 0)),
        ],
        out_shape=[
            jax.ShapeDtypeStruct((n, D), jnp.float32),
            jax.ShapeDtypeStruct((n, 128), jnp.float32),
        ],
    )(x, w, a_mat)
    return h, s[:, :H]


def _ln_kernel(x_ref, bias_ref, g_ref, b_ref, o_ref):
    x = x_ref[...] + bias_ref[...]
    mu = jnp.mean(x, axis=-1, keepdims=True)
    var = jnp.mean((x - mu) * (x - mu), axis=-1, keepdims=True)
    y = (x - mu) * jax.lax.rsqrt(var + 1e-5) * g_ref[...] + b_ref[...]
    o_ref[...] = jnp.maximum(y, 0.0)


def _ln_relu(x, bias, g, b):
    n = x.shape[0]
    return pl.pallas_call(
        _ln_kernel,
        grid=(n // TILE,),
        in_specs=[
            pl.BlockSpec((TILE, D), lambda i: (i, 0)),
            pl.BlockSpec((1, D), lambda i: (0, 0)),
            pl.BlockSpec((1, D), lambda i: (0, 0)),
            pl.BlockSpec((1, D), lambda i: (0, 0)),
        ],
        out_specs=pl.BlockSpec((TILE, D), lambda i: (i, 0)),
        out_shape=jax.ShapeDtypeStruct((n, D), jnp.float32),
    )(x, bias.reshape(1, D), g.reshape(1, D), b.reshape(1, D))


def _a_mat(a):
    eye = jnp.eye(H, dtype=jnp.float32)
    m = (a[:, :, None] * eye[:, None, :]).reshape(D, H)
    return jnp.pad(m, ((0, 0), (0, 128 - H)))


def _gat(h_src, s_src, x_dst_pad, edge_index, s_e, p):
    src = edge_index[0]
    dst = edge_index[1]
    _, s_dst = _proj(x_dst_pad, p["W_dst"], _a_mat(p["a_dst"]))
    alpha = s_src[src] + s_dst[:N_NODE][dst] + s_e
    alpha = jnp.where(alpha >= 0, alpha, 0.2 * alpha)
    m = jax.ops.segment_max(alpha, dst, num_segments=N_NODE)
    m = jnp.where(jnp.isfinite(m), m, 0.0)
    ex = jnp.exp(alpha - m[dst])
    den = jax.ops.segment_sum(ex, dst, num_segments=N_NODE)
    w = ex / (den[dst] + 1e-16)
    msg = h_src[src] * jnp.repeat(w, C, axis=1)
    return jax.ops.segment_sum(msg, dst, num_segments=N_NODE)


def kernel(x_user, x_item, edge_attr_u2i, edge_attr_i2u, params, edge_index_u2i, edge_index_i2u):
    pad_n = ((0, N_PAD - N_NODE), (0, 0))
    ea_u2i = jnp.pad(edge_attr_u2i, ((0, 0), (0, 128 - edge_attr_u2i.shape[1])))
    ea_i2u = jnp.pad(edge_attr_i2u, ((0, 0), (0, 128 - edge_attr_i2u.shape[1])))
    xu = jnp.pad(x_user, pad_n)
    xi = jnp.pad(x_item, pad_n)
    for lp in params["layers"]:
        pu, pi = lp["u2i"], lp["i2u"]
        h_u, s_u = _proj(xu, pu["W_src"], _a_mat(pu["a_src"]))
        h_i, s_i = _proj(xi, pi["W_src"], _a_mat(pi["a_src"]))
        we_u = jnp.pad(pu["W_edge"], ((0, 128 - pu["W_edge"].shape[0]), (0, 0)))
        we_i = jnp.pad(pi["W_edge"], ((0, 128 - pi["W_edge"].shape[0]), (0, 0)))
        _, se_u = _proj(ea_u2i, we_u, _a_mat(pu["a_edge"]))
        _, se_i = _proj(ea_i2u, we_i, _a_mat(pi["a_edge"]))
        agg_item = _gat(h_u[:N_NODE], s_u[:N_NODE], xi, edge_index_u2i, se_u, pu)
        agg_user = _gat(h_i[:N_NODE], s_i[:N_NODE], xu, edge_index_i2u, se_i, pi)
        xu = _ln_relu(jnp.pad(agg_user, pad_n), pi["bias"], lp["g_user"], lp["b_user"])
        xi = _ln_relu(jnp.pad(agg_item, pad_n), pu["bias"], lp["g_item"], lp["b_item"])
    return jnp.concatenate([xu[:N_NODE], xi[:N_NODE]], axis=0)
